# baseline (speedup 1.0000x reference)
import functools

import jax
import jax.numpy as jnp
from jax import lax
from jax.experimental import pallas as pl
from jax.experimental.pallas import tpu as pltpu
from jax.experimental.pallas import tpu_sc as plsc

N_USER = 2000
N_ITEM = 8000
N_NODES = N_USER + N_ITEM
DIM = 128
N_EDGES = 320000

NS = 16
CHUNK = 128
TOTAL_CHUNKS = N_EDGES // CHUNK
ACC_ROWS = 10240
ROWS_PER_SUB = ACC_ROWS // NS
ZROWS = 64

_sc_mesh = plsc.VectorSubcoreMesh(
    core_axis_name="c", subcore_axis_name="s", num_cores=1)


@functools.partial(
    pl.kernel,
    out_type=jax.ShapeDtypeStruct((ACC_ROWS, DIM), jnp.float32),
    mesh=_sc_mesh,
    scratch_types=(
        [pltpu.VMEM((2, CHUNK), jnp.int32)] * 4
        + [pltpu.VMEM((CHUNK, DIM), jnp.float32)] * 2
        + [
            pltpu.VMEM((ZROWS, DIM), jnp.float32),
            pltpu.VMEM_SHARED((ACC_ROWS, DIM), jnp.float32),
        ]
        + [pltpu.SemaphoreType.DMA] * 7
    ),
)
def _sc_aggregate(xn_hbm, edge_hbm, out_hbm,
                  e0, e1, e2, e3, rows0, rows1,
                  zero_v, acc_sh, si0, si1, si2, si3, sg0, sg1, sz):
    idx_s = (e0, e1, e2, e3)
    rows_s = (rows0, rows1)
    sem_i = (si0, si1, si2, si3)
    sem_g = (sg0, sg1)
    sid = lax.axis_index("s")

    base_c = sid * TOTAL_CHUNKS // NS
    n_chunks = (sid + 1) * TOTAL_CHUNKS // NS - base_c
    base = base_c * CHUNK

    def _issue_idx(i, slot):
        off = base + i * CHUNK
        pltpu.async_copy(edge_hbm.at[:, pl.ds(off, CHUNK)], idx_s[slot],
                         sem_i[slot])

    def _wait_idx(i, slot):
        off = base + i * CHUNK
        pltpu.make_async_copy(edge_hbm.at[:, pl.ds(off, CHUNK)], idx_s[slot],
                              sem_i[slot]).wait()

    _issue_idx(0, 0)
    _issue_idx(1, 1)

    @pl.loop(0, ZROWS)
    def _(r):
        @pl.loop(0, DIM, step=16)
        def _(q):
            zero_v[pl.ds(r, 1), pl.ds(q, 16)] = jnp.zeros((1, 16), jnp.float32)

    _wait_idx(0, 0)
    pltpu.async_copy(xn_hbm.at[idx_s[0].at[0]], rows_s[0], sem_g[0])

    @pl.loop(0, ROWS_PER_SUB, step=ZROWS)
    def _(r):
        pltpu.async_copy(
            zero_v, acc_sh.at[pl.ds(sid * ROWS_PER_SUB + r, ZROWS)], sz)

    @pl.loop(0, ROWS_PER_SUB, step=ZROWS)
    def _(r):
        pltpu.make_async_copy(
            zero_v, acc_sh.at[pl.ds(sid * ROWS_PER_SUB + r, ZROWS)], sz).wait()

    plsc.subcore_barrier()


    def _quad(q, carry):
        for p in range(4):
            i = 4 * q + p
            nxt = (p + 1) % 4
            pre = (p + 2) % 4

            @pl.when(i + 1 < n_chunks)
            def _():
                _wait_idx(i + 1, nxt)
                pltpu.async_copy(xn_hbm.at[idx_s[nxt].at[0]],
                                 rows_s[(p + 1) % 2], sem_g[(p + 1) % 2])

            @pl.when(i + 2 < n_chunks)
            def _():
                _issue_idx(i + 2, pre)

            @pl.when(i < n_chunks)
            def _():
                pltpu.make_async_copy(
                    xn_hbm.at[idx_s[p].at[0]], rows_s[p % 2],
                    sem_g[p % 2]).wait()
                pltpu.sync_copy(rows_s[p % 2], acc_sh.at[idx_s[p].at[1]],
                                add=True)
        return carry

    lax.fori_loop(0, (n_chunks + 3) // 4, _quad, 0)

    plsc.subcore_barrier()

    pltpu.sync_copy(acc_sh.at[pl.ds(sid * ROWS_PER_SUB, ROWS_PER_SUB)],
                    out_hbm.at[pl.ds(sid * ROWS_PER_SUB, ROWS_PER_SUB)])


def _normalize_body(pref_ref, feat_ref, o_ref):
    i = pl.program_id(0)
    x = jnp.where(i < N_USER // _ROWB, pref_ref[...], feat_ref[...])
    s = jnp.sum(x * x, axis=1, keepdims=True)
    o_ref[...] = x * lax.rsqrt(jnp.maximum(s, 1e-24))


def _combine_body(p_ref, xn_ref, w_ref, o_ref):
    o_ref[...] = (
        jnp.dot(p_ref[...].astype(jnp.bfloat16),
                w_ref[...].astype(jnp.bfloat16),
                preferred_element_type=jnp.float32)
        + xn_ref[...]
    )


_ROWB = 1000


def kernel(edge_index, features, preference, W):
    n_pref_blocks = N_USER // _ROWB

    xn = pl.pallas_call(
        _normalize_body,
        out_shape=jax.ShapeDtypeStruct((N_NODES, DIM), jnp.float32),
        grid=(N_NODES // _ROWB,),
        in_specs=[
            pl.BlockSpec((_ROWB, DIM),
                         lambda i: (jnp.minimum(i, n_pref_blocks - 1), 0)),
            pl.BlockSpec((_ROWB, DIM),
                         lambda i: (jnp.maximum(i - n_pref_blocks, 0), 0)),
        ],
        out_specs=pl.BlockSpec((_ROWB, DIM), lambda i: (i, 0)),
        compiler_params=pltpu.CompilerParams(
            dimension_semantics=("parallel",)),
    )(preference, features)

    part = _sc_aggregate(xn, edge_index)

    x_hat = pl.pallas_call(
        _combine_body,
        out_shape=jax.ShapeDtypeStruct((N_NODES, DIM), jnp.float32),
        grid=(N_NODES // _ROWB,),
        in_specs=[
            pl.BlockSpec((_ROWB, DIM), lambda i: (i, 0)),
            pl.BlockSpec((_ROWB, DIM), lambda i: (i, 0)),
            pl.BlockSpec((DIM, DIM), lambda i: (0, 0)),
        ],
        out_specs=pl.BlockSpec((_ROWB, DIM), lambda i: (i, 0)),
        compiler_params=pltpu.CompilerParams(
            dimension_semantics=("parallel",)),
    )(part, xn, W)

    return (x_hat, preference)


# speedup vs baseline: 1.0266x; 1.0266x over previous
import functools

import jax
import jax.numpy as jnp
from jax import lax
from jax.experimental import pallas as pl
from jax.experimental.pallas import tpu as pltpu
from jax.experimental.pallas import tpu_sc as plsc

N_USER = 2000
N_ITEM = 8000
N_NODES = N_USER + N_ITEM
DIM = 128
N_EDGES = 320000

NS = 16
CHUNK = 128
TOTAL_CHUNKS = N_EDGES // CHUNK
ACC_ROWS = 10240
ROWS_PER_SUB = ACC_ROWS // NS
ZROWS = 64

_sc_mesh = plsc.VectorSubcoreMesh(
    core_axis_name="c", subcore_axis_name="s", num_cores=1)


@functools.partial(
    pl.kernel,
    out_type=jax.ShapeDtypeStruct((ACC_ROWS, DIM), jnp.float32),
    mesh=_sc_mesh,
    scratch_types=(
        [pltpu.VMEM((2, CHUNK), jnp.int32)] * 4
        + [pltpu.VMEM((CHUNK, DIM), jnp.float32)] * 2
        + [
            pltpu.VMEM((ZROWS, DIM), jnp.float32),
            pltpu.VMEM_SHARED((ACC_ROWS, DIM), jnp.float32),
        ]
        + [pltpu.SemaphoreType.DMA] * 7
    ),
)
def _sc_aggregate(xn_hbm, edge_hbm, out_hbm,
                  e0, e1, e2, e3, rows0, rows1,
                  zero_v, acc_sh, si0, si1, si2, si3, sg0, sg1, sz):
    idx_s = (e0, e1, e2, e3)
    rows_s = (rows0, rows1)
    sem_i = (si0, si1, si2, si3)
    sem_g = (sg0, sg1)
    sid = lax.axis_index("s")

    base_c = sid * TOTAL_CHUNKS // NS
    n_chunks = (sid + 1) * TOTAL_CHUNKS // NS - base_c
    base = base_c * CHUNK

    def _issue_idx(i, slot):
        off = base + i * CHUNK
        pltpu.async_copy(edge_hbm.at[:, pl.ds(off, CHUNK)], idx_s[slot],
                         sem_i[slot])

    def _wait_idx(i, slot):
        off = base + i * CHUNK
        pltpu.make_async_copy(edge_hbm.at[:, pl.ds(off, CHUNK)], idx_s[slot],
                              sem_i[slot]).wait()

    _issue_idx(0, 0)
    _issue_idx(1, 1)

    @pl.loop(0, ZROWS)
    def _(r):
        @pl.loop(0, DIM, step=16)
        def _(q):
            zero_v[pl.ds(r, 1), pl.ds(q, 16)] = jnp.zeros((1, 16), jnp.float32)

    _wait_idx(0, 0)
    pltpu.async_copy(xn_hbm.at[idx_s[0].at[0]], rows_s[0], sem_g[0])

    @pl.loop(0, ROWS_PER_SUB, step=ZROWS)
    def _(r):
        pltpu.async_copy(
            zero_v, acc_sh.at[pl.ds(sid * ROWS_PER_SUB + r, ZROWS)], sz)

    @pl.loop(0, ROWS_PER_SUB, step=ZROWS)
    def _(r):
        pltpu.make_async_copy(
            zero_v, acc_sh.at[pl.ds(sid * ROWS_PER_SUB + r, ZROWS)], sz).wait()

    plsc.subcore_barrier()


    def _quad(q, carry):
        for p in range(4):
            i = 4 * q + p
            nxt = (p + 1) % 4
            pre = (p + 2) % 4

            @pl.when(i + 1 < n_chunks)
            def _():
                _wait_idx(i + 1, nxt)
                pltpu.async_copy(xn_hbm.at[idx_s[nxt].at[0]],
                                 rows_s[(p + 1) % 2], sem_g[(p + 1) % 2])

            @pl.when(i + 2 < n_chunks)
            def _():
                _issue_idx(i + 2, pre)

            @pl.when(i < n_chunks)
            def _():
                pltpu.make_async_copy(
                    xn_hbm.at[idx_s[p].at[0]], rows_s[p % 2],
                    sem_g[p % 2]).wait()
                pltpu.sync_copy(rows_s[p % 2], acc_sh.at[idx_s[p].at[1]],
                                add=True)
        return carry

    lax.fori_loop(0, (n_chunks + 3) // 4, _quad, 0)

    plsc.subcore_barrier()

    pltpu.sync_copy(acc_sh.at[pl.ds(sid * ROWS_PER_SUB, ROWS_PER_SUB)],
                    out_hbm.at[pl.ds(sid * ROWS_PER_SUB, ROWS_PER_SUB)])


def _normalize_body(pref_ref, feat_ref, o_ref):
    i = pl.program_id(0)
    x = jnp.where(i < N_USER // _ROWB, pref_ref[...], feat_ref[...])
    s = jnp.sum(x * x, axis=1, keepdims=True)
    o_ref[...] = x * lax.rsqrt(jnp.maximum(s, 1e-24))


def _combine_body(p_ref, xn_ref, w_ref, o_ref):
    o_ref[...] = (
        jnp.dot(p_ref[...].astype(jnp.bfloat16),
                w_ref[...].astype(jnp.bfloat16),
                preferred_element_type=jnp.float32)
        + xn_ref[...]
    )


_ROWB = 2000


def kernel(edge_index, features, preference, W):
    n_pref_blocks = N_USER // _ROWB

    xn = pl.pallas_call(
        _normalize_body,
        out_shape=jax.ShapeDtypeStruct((N_NODES, DIM), jnp.float32),
        grid=(N_NODES // _ROWB,),
        in_specs=[
            pl.BlockSpec((_ROWB, DIM),
                         lambda i: (jnp.minimum(i, n_pref_blocks - 1), 0)),
            pl.BlockSpec((_ROWB, DIM),
                         lambda i: (jnp.maximum(i - n_pref_blocks, 0), 0)),
        ],
        out_specs=pl.BlockSpec((_ROWB, DIM), lambda i: (i, 0)),
    )(preference, features)

    part = _sc_aggregate(xn, edge_index)

    x_hat = pl.pallas_call(
        _combine_body,
        out_shape=jax.ShapeDtypeStruct((N_NODES, DIM), jnp.float32),
        grid=(N_NODES // _ROWB,),
        in_specs=[
            pl.BlockSpec((_ROWB, DIM), lambda i: (i, 0)),
            pl.BlockSpec((_ROWB, DIM), lambda i: (i, 0)),
            pl.BlockSpec((DIM, DIM), lambda i: (0, 0)),
        ],
        out_specs=pl.BlockSpec((_ROWB, DIM), lambda i: (i, 0)),
    )(part, xn, W)

    return (x_hat, preference)
